# Initial kernel scaffold; baseline (speedup 1.0000x reference)
import functools
import math

import jax
import jax.numpy as jnp
from jax import lax
from jax.experimental import pallas as pl
from jax.experimental.pallas import tpu as pltpu
from jax.experimental.pallas import tpu_sc as plsc

N = 10000
E = 320000
H = 128
NPAD = 10240
NC, NS = 2, 16
NW = NC * NS
EPW = E // NW
CH = 125
NCHUNK = EPW // CH
BPW = NPAD // NW
RPS = NPAD // NS

_MESH = dict(core_axis_name="c", subcore_axis_name="s")


def _wid():
    return lax.axis_index("s") * NC + lax.axis_index("c")


def _k0_body(emb_hbm, idsq_hbm, x_hbm, idx_v, rows_v, sem):
    w = _wid()
    pltpu.sync_copy(idsq_hbm.at[w], idx_v)
    for j in range(5):
        pltpu.async_copy(emb_hbm.at[idx_v.at[j]],
                         rows_v.at[pl.ds(64 * j, 64)], sem).wait()
    pltpu.sync_copy(rows_v, x_hbm.at[pl.ds(w * BPW, BPW)])


def _k0(emb, idsq):
    return pl.kernel(
        _k0_body,
        out_type=jax.ShapeDtypeStruct((NPAD, H), jnp.float32),
        mesh=plsc.VectorSubcoreMesh(**_MESH),
        scratch_types=[
            pltpu.VMEM((5, 64), jnp.int32),
            pltpu.VMEM((BPW, H), jnp.float32),
            pltpu.SemaphoreType.DMA,
        ],
    )(emb, idsq)


def _k1_body(ssrc_hbm, sdst_hbm, src_hbm, dst_hbm, scores_hbm, maxes_hbm,
             ssrc_v, sdst_v, src_v, dst_v, sc_v, mx_v, sem):
    w = _wid()
    pltpu.sync_copy(ssrc_hbm, ssrc_v)
    pltpu.sync_copy(sdst_hbm, sdst_v)
    pltpu.sync_copy(src_hbm.at[pl.ds(w * EPW, EPW)], src_v)
    pltpu.sync_copy(dst_hbm.at[pl.ds(w * EPW, EPW)], dst_v)

    def body(k, mx):
        o = pl.multiple_of(16 * k, 16)
        vs = plsc.load_gather(ssrc_v, [src_v[pl.ds(o, 16)]])
        vd = plsc.load_gather(sdst_v, [dst_v[pl.ds(o, 16)]])
        s = vs + vd
        s = jnp.where(s >= 0.0, s, 0.2 * s)
        sc_v[pl.ds(o, 16)] = s
        return jnp.maximum(mx, s)

    mx = lax.fori_loop(0, EPW // 16, body,
                       jnp.full((16,), -3.0e38, jnp.float32))
    mx_v[...] = mx
    pltpu.sync_copy(sc_v, scores_hbm.at[pl.ds(w * EPW, EPW)])
    pltpu.sync_copy(mx_v, maxes_hbm.at[pl.ds(16 * w, 16)])


def _k1(ssrc, sdst, src, dst):
    return pl.kernel(
        _k1_body,
        out_type=(jax.ShapeDtypeStruct((E,), jnp.float32),
                  jax.ShapeDtypeStruct((16 * NW,), jnp.float32)),
        mesh=plsc.VectorSubcoreMesh(**_MESH),
        scratch_types=[
            pltpu.VMEM((NPAD,), jnp.float32),
            pltpu.VMEM((NPAD,), jnp.float32),
            pltpu.VMEM((EPW,), jnp.int32),
            pltpu.VMEM((EPW,), jnp.int32),
            pltpu.VMEM((EPW,), jnp.float32),
            pltpu.VMEM((16,), jnp.float32),
            pltpu.SemaphoreType.DMA,
        ],
    )(ssrc, sdst, src, dst)


def _k2_body(scores_hbm, maxes_hbm, srcq_hbm, dstq_hbm, h_hbm,
             agg_out, den_out,
             agg_sh, den_sh,
             src_c, dst_c, sc_v, attn_v, rows_v, maxm_v, zrow_v, zden_v, sem):
    c = lax.axis_index("c")
    s = lax.axis_index("s")
    w = s * NC + c

    z16 = jnp.zeros((16,), jnp.float32)
    for i in range(16):
        for j in range(H // 16):
            zrow_v[i, pl.ds(16 * j, 16)] = z16
    for k in range(RPS // 16):
        zden_v[pl.ds(16 * k, 16)] = z16
    for k in range(RPS // 16):
        pltpu.sync_copy(zrow_v, agg_sh.at[pl.ds(RPS * s + 16 * k, 16)])
    pltpu.sync_copy(zden_v, den_sh.at[pl.ds(RPS * s, RPS)])
    plsc.subcore_barrier()

    pltpu.sync_copy(maxes_hbm, maxm_v)
    pltpu.sync_copy(scores_hbm.at[pl.ds(w * EPW, EPW)], sc_v)
    pltpu.sync_copy(srcq_hbm.at[w], src_c)
    pltpu.sync_copy(dstq_hbm.at[w], dst_c)

    def maxbody(k, m):
        o = pl.multiple_of(16 * k, 16)
        return jnp.maximum(m, maxm_v[pl.ds(o, 16)])

    m16 = lax.fori_loop(0, NW, maxbody, jnp.full((16,), -3.0e38, jnp.float32))
    M = jnp.max(m16)

    def expbody(k, _):
        o = pl.multiple_of(16 * k, 16)
        attn_v[pl.ds(o, 16)] = jnp.exp(sc_v[pl.ds(o, 16)] - M)
        return 0

    lax.fori_loop(0, EPW // 16, expbody, 0)

    def chunk(ci, _):
        pltpu.async_copy(h_hbm.at[src_c.at[ci]], rows_v, sem).wait()

        def edge(i, _):
            a = plsc.load_gather(attn_v, [lax.broadcast(ci * CH + i, (16,))])
            for j in range(H // 16):
                rows_v[i, pl.ds(16 * j, 16)] = rows_v[i, pl.ds(16 * j, 16)] * a
            return 0

        lax.fori_loop(0, CH, edge, 0)
        pltpu.sync_copy(rows_v, agg_sh.at[dst_c.at[ci]], add=True)
        pltpu.sync_copy(attn_v.at[pl.ds(ci * CH, CH)],
                        den_sh.at[dst_c.at[ci]], add=True)
        return 0

    lax.fori_loop(0, NCHUNK, chunk, 0)
    plsc.subcore_barrier()

    for k in range(RPS // 16):
        pltpu.sync_copy(agg_sh.at[pl.ds(RPS * s + 16 * k, 16)], zrow_v)
        pltpu.sync_copy(zrow_v, agg_out.at[c].at[pl.ds(RPS * s + 16 * k, 16)])
    pltpu.sync_copy(den_sh.at[pl.ds(RPS * s, RPS)], zden_v)
    pltpu.sync_copy(zden_v, den_out.at[c].at[pl.ds(RPS * s, RPS)])


def _k2(scores, maxes, srcq, dstq, h):
    return pl.kernel(
        _k2_body,
        out_type=(jax.ShapeDtypeStruct((NC, NPAD, H), jnp.float32),
                  jax.ShapeDtypeStruct((NC, NPAD), jnp.float32)),
        mesh=plsc.VectorSubcoreMesh(**_MESH),
        scratch_types=[
            pltpu.VMEM_SHARED((NPAD, H), jnp.float32),
            pltpu.VMEM_SHARED((NPAD,), jnp.float32),
            pltpu.VMEM((NCHUNK, CH), jnp.int32),
            pltpu.VMEM((NCHUNK, CH), jnp.int32),
            pltpu.VMEM((EPW,), jnp.float32),
            pltpu.VMEM((EPW,), jnp.float32),
            pltpu.VMEM((CH, H), jnp.float32),
            pltpu.VMEM((16 * NW,), jnp.float32),
            pltpu.VMEM((16, H), jnp.float32),
            pltpu.VMEM((RPS,), jnp.float32),
            pltpu.SemaphoreType.DMA,
        ],
    )(scores, maxes, srcq, dstq, h)


_BLK = 1024
_GRID = NPAD // _BLK


def _tc1_body(x_ref, wp_ref, wep_ref, h_ref, s2_ref):
    h = jnp.dot(x_ref[...], wp_ref[...], preferred_element_type=jnp.float32)
    h_ref[...] = h
    s2_ref[...] = lax.dot_general(h, wep_ref[...], (((1,), (1,)), ((), ())),
                                  preferred_element_type=jnp.float32)


def _tc1(x, Wp, Wepair):
    return pl.pallas_call(
        _tc1_body,
        grid=(_GRID,),
        in_specs=[
            pl.BlockSpec((_BLK, H), lambda i: (i, 0)),
            pl.BlockSpec((H, H), lambda i: (0, 0)),
            pl.BlockSpec((2, H), lambda i: (0, 0)),
        ],
        out_specs=[
            pl.BlockSpec((_BLK, H), lambda i: (i, 0)),
            pl.BlockSpec((_BLK, 2), lambda i: (i, 0)),
        ],
        out_shape=[
            jax.ShapeDtypeStruct((NPAD, H), jnp.float32),
            jax.ShapeDtypeStruct((NPAD, 2), jnp.float32),
        ],
    )(x, Wp, Wepair)


def _tc2_body(agg_ref, den_ref, x_ref, g_ref, b_ref, o_ref):
    a = agg_ref[0] + agg_ref[1]
    d = den_ref[0] + den_ref[1] + 1e-6
    y = a / d[:, None] + x_ref[...]
    mu = jnp.mean(y, axis=-1, keepdims=True)
    var = jnp.mean((y - mu) ** 2, axis=-1, keepdims=True)
    y = (y - mu) / jnp.sqrt(var + 1e-5) * g_ref[...] + b_ref[...]
    o_ref[...] = 0.5 * y * (1.0 + lax.erf(y / math.sqrt(2.0)))


def _tc2(agg, den, x, g, b):
    return pl.pallas_call(
        _tc2_body,
        grid=(_GRID,),
        in_specs=[
            pl.BlockSpec((NC, _BLK, H), lambda i: (0, i, 0)),
            pl.BlockSpec((NC, _BLK), lambda i: (0, i)),
            pl.BlockSpec((_BLK, H), lambda i: (i, 0)),
            pl.BlockSpec((1, H), lambda i: (0, 0)),
            pl.BlockSpec((1, H), lambda i: (0, 0)),
        ],
        out_specs=pl.BlockSpec((_BLK, H), lambda i: (i, 0)),
        out_shape=jax.ShapeDtypeStruct((NPAD, H), jnp.float32),
    )(agg, den, x, g, b)


def _tc3_body(x_ref, wq_ref, bq_ref, swr_ref, mask_ref, vec_ref, w_ref):
    x = x_ref[...]
    proj = jnp.tanh(jnp.dot(x, wq_ref[...],
                            preferred_element_type=jnp.float32) + bq_ref[...])
    sc = lax.dot_general(proj, swr_ref[...], (((1,), (1,)), ((), ())),
                         preferred_element_type=jnp.float32)[:, 0]
    sc = jnp.where(mask_ref[...] > 0.0, sc, -jnp.inf)
    m = jnp.max(sc)
    e = jnp.exp(sc - m)
    wgt = e / jnp.sum(e)
    wgt = jnp.where(jnp.isnan(wgt), 0.0, wgt)
    w_ref[...] = wgt
    gv = lax.dot_general(wgt, x, (((0,), (0,)), ((), ())),
                         preferred_element_type=jnp.float32)
    vec_ref[...] = gv[None, :]


def _tc3(x, Wq, bq, swr, mask):
    return pl.pallas_call(
        _tc3_body,
        out_shape=[
            jax.ShapeDtypeStruct((1, H), jnp.float32),
            jax.ShapeDtypeStruct((NPAD,), jnp.float32),
        ],
    )(x, Wq, bq, swr, mask)


def _gat_layer(x, src, dst, srcq, dstq, Wp, Wepair, g, b):
    h, s2 = _tc1(x, Wp, Wepair)
    scores, maxes = _k1(s2[:, 0], s2[:, 1], src, dst)
    agg, den = _k2(scores, maxes, srcq, dstq, h)
    return _tc2(agg, den, x, g, b)


def kernel(node_ids, edge_index, emb, W1p, W1e, g1, b1, W2p, W2e, g2, b2,
           Wq, bq, sw):
    f32 = jnp.float32
    ids = jnp.concatenate([node_ids.astype(jnp.int32),
                           jnp.zeros((NPAD - N,), jnp.int32)])
    idsq = ids.reshape(NW, 5, 64)
    src = edge_index[0].astype(jnp.int32)
    dst = edge_index[1].astype(jnp.int32)
    srcq = src.reshape(NW, NCHUNK, CH)
    dstq = dst.reshape(NW, NCHUNK, CH)
    mask = (ids != 0).astype(f32)

    x = _k0(emb, idsq)
    We1 = W1e[:, 0].reshape(2, H)
    We2 = W2e[:, 0].reshape(2, H)
    x = _gat_layer(x, src, dst, srcq, dstq, W1p, We1,
                   g1.reshape(1, H), b1.reshape(1, H))
    x = _gat_layer(x, src, dst, srcq, dstq, W2p, We2,
                   g2.reshape(1, H), b2.reshape(1, H))

    vec, wts = _tc3(x, Wq, bq.reshape(1, H), sw[:, 0].reshape(1, H), mask)
    return (vec, wts[:N], jnp.ones((1,), f32))


# speedup vs baseline: 7.9907x; 7.9907x over previous
import functools
import math

import jax
import jax.numpy as jnp
from jax import lax
from jax.experimental import pallas as pl
from jax.experimental.pallas import tpu as pltpu
from jax.experimental.pallas import tpu_sc as plsc

N = 10000
E = 320000
H = 128
NPAD = 10240
NC, NS = 2, 16
NW = NC * NS
EPW = E // NW
CH = 80
NCHUNK = EPW // CH
BPW = NPAD // NW
RPS = NPAD // NS

_MESH = dict(core_axis_name="c", subcore_axis_name="s")


def _wid():
    return lax.axis_index("s") * NC + lax.axis_index("c")


def _k0_body(emb_hbm, idsq_hbm, x_hbm, idx_v, rows_v, sem):
    w = _wid()
    pltpu.sync_copy(idsq_hbm.at[w], idx_v)
    for j in range(5):
        pltpu.async_copy(emb_hbm.at[idx_v.at[j]],
                         rows_v.at[pl.ds(64 * j, 64)], sem).wait()
    pltpu.sync_copy(rows_v, x_hbm.at[pl.ds(w * BPW, BPW)])


def _k0(emb, idsq):
    return pl.kernel(
        _k0_body,
        out_type=jax.ShapeDtypeStruct((NPAD, H), jnp.float32),
        mesh=plsc.VectorSubcoreMesh(**_MESH),
        compiler_params=pltpu.CompilerParams(needs_layout_passes=False),
        scratch_types=[
            pltpu.VMEM((5, 64), jnp.int32),
            pltpu.VMEM((BPW, H), jnp.float32),
            pltpu.SemaphoreType.DMA,
        ],
    )(emb, idsq)


def _k1_body(ssrc_hbm, sdst_hbm, src_hbm, dst_hbm, scores_hbm, maxes_hbm,
             ssrc_v, sdst_v, src_v, dst_v, sc_v, mx_v, sem):
    w = _wid()
    pltpu.sync_copy(ssrc_hbm, ssrc_v)
    pltpu.sync_copy(sdst_hbm, sdst_v)
    pltpu.sync_copy(src_hbm.at[pl.ds(w * EPW, EPW)], src_v)
    pltpu.sync_copy(dst_hbm.at[pl.ds(w * EPW, EPW)], dst_v)

    def body(k, mx):
        o = pl.multiple_of(16 * k, 16)
        vs = plsc.load_gather(ssrc_v, [src_v[pl.ds(o, 16)]])
        vd = plsc.load_gather(sdst_v, [dst_v[pl.ds(o, 16)]])
        s = vs + vd
        s = jnp.where(s >= 0.0, s, 0.2 * s)
        sc_v[pl.ds(o, 16)] = s
        return jnp.maximum(mx, s)

    mx = lax.fori_loop(0, EPW // 16, body,
                       jnp.full((16,), -3.0e38, jnp.float32))
    mx_v[...] = mx
    pltpu.sync_copy(sc_v, scores_hbm.at[pl.ds(w * EPW, EPW)])
    pltpu.sync_copy(mx_v, maxes_hbm.at[pl.ds(16 * w, 16)])


def _k1(ssrc, sdst, src, dst):
    return pl.kernel(
        _k1_body,
        out_type=(jax.ShapeDtypeStruct((E,), jnp.float32),
                  jax.ShapeDtypeStruct((16 * NW,), jnp.float32)),
        mesh=plsc.VectorSubcoreMesh(**_MESH),
        compiler_params=pltpu.CompilerParams(needs_layout_passes=False),
        scratch_types=[
            pltpu.VMEM((NPAD,), jnp.float32),
            pltpu.VMEM((NPAD,), jnp.float32),
            pltpu.VMEM((EPW,), jnp.int32),
            pltpu.VMEM((EPW,), jnp.int32),
            pltpu.VMEM((EPW,), jnp.float32),
            pltpu.VMEM((16,), jnp.float32),
            pltpu.SemaphoreType.DMA,
        ],
    )(ssrc, sdst, src, dst)


def _k2_body(scores_hbm, maxes_hbm, srcq_hbm, dstq_hbm, h_hbm,
             agg_out, den_out,
             agg_sh, den_sh,
             idxs_v, idxd_v, sc_v, rows_v, maxm_v, zrow_v, zden_v, sem):
    c = lax.axis_index("c")
    s = lax.axis_index("s")
    w = s * NC + c

    z16 = jnp.zeros((16,), jnp.float32)
    for i in range(16):
        for j in range(H // 16):
            zrow_v[i, pl.ds(16 * j, 16)] = z16
    for k in range(RPS // 16):
        zden_v[pl.ds(16 * k, 16)] = z16
    for k in range(RPS // 16):
        pltpu.sync_copy(zrow_v, agg_sh.at[pl.ds(RPS * s + 16 * k, 16)])
    pltpu.sync_copy(zden_v, den_sh.at[pl.ds(RPS * s, RPS)])
    plsc.subcore_barrier()

    pltpu.sync_copy(maxes_hbm, maxm_v)
    pltpu.sync_copy(scores_hbm.at[pl.ds(w * EPW, EPW)], sc_v)

    def maxbody(k, m):
        o = pl.multiple_of(16 * k, 16)
        return jnp.maximum(m, maxm_v[pl.ds(o, 16)])

    m16 = lax.fori_loop(0, NW, maxbody, jnp.full((16,), -3.0e38, jnp.float32))
    M = jnp.max(m16)

    def expbody(k, _):
        o = pl.multiple_of(16 * k, 16)
        sc_v[pl.ds(o, 16)] = jnp.exp(sc_v[pl.ds(o, 16)] - M)
        return 0

    lax.fori_loop(0, EPW // 16, expbody, 0)

    def chunk(ci, _):
        pltpu.sync_copy(srcq_hbm.at[w].at[ci], idxs_v)
        pltpu.sync_copy(dstq_hbm.at[w].at[ci], idxd_v)
        pltpu.async_copy(h_hbm.at[idxs_v], rows_v, sem).wait()

        def edge(i, _):
            a = plsc.load_gather(sc_v, [lax.broadcast(ci * CH + i, (16,))])
            for j in range(H // 16):
                rows_v[i, pl.ds(16 * j, 16)] = rows_v[i, pl.ds(16 * j, 16)] * a
            return 0

        lax.fori_loop(0, CH, edge, 0)
        pltpu.sync_copy(rows_v, agg_sh.at[idxd_v], add=True)
        pltpu.sync_copy(sc_v.at[pl.ds(ci * CH, CH)],
                        den_sh.at[idxd_v], add=True)
        return 0

    lax.fori_loop(0, NCHUNK, chunk, 0)
    plsc.subcore_barrier()

    for k in range(RPS // 16):
        pltpu.sync_copy(agg_sh.at[pl.ds(RPS * s + 16 * k, 16)], zrow_v)
        pltpu.sync_copy(zrow_v, agg_out.at[c].at[pl.ds(RPS * s + 16 * k, 16)])
    pltpu.sync_copy(den_sh.at[pl.ds(RPS * s, RPS)], zden_v)
    pltpu.sync_copy(zden_v, den_out.at[c].at[pl.ds(RPS * s, RPS)])


def _k2(scores, maxes, srcq, dstq, h):
    return pl.kernel(
        _k2_body,
        out_type=(jax.ShapeDtypeStruct((NC, NPAD, H), jnp.float32),
                  jax.ShapeDtypeStruct((NC, NPAD), jnp.float32)),
        mesh=plsc.VectorSubcoreMesh(**_MESH),
        compiler_params=pltpu.CompilerParams(needs_layout_passes=False),
        scratch_types=[
            pltpu.VMEM_SHARED((NPAD, H), jnp.float32),
            pltpu.VMEM_SHARED((NPAD,), jnp.float32),
            pltpu.VMEM((CH,), jnp.int32),
            pltpu.VMEM((CH,), jnp.int32),
            pltpu.VMEM((EPW,), jnp.float32),
            pltpu.VMEM((CH, H), jnp.float32),
            pltpu.VMEM((16 * NW,), jnp.float32),
            pltpu.VMEM((16, H), jnp.float32),
            pltpu.VMEM((RPS,), jnp.float32),
            pltpu.SemaphoreType.DMA,
        ],
    )(scores, maxes, srcq, dstq, h)


_BLK = 1024
_GRID = NPAD // _BLK


def _tc1_body(x_ref, wp_ref, wep_ref, h_ref, s2_ref):
    h = jnp.dot(x_ref[...], wp_ref[...], preferred_element_type=jnp.float32)
    h_ref[...] = h
    s2_ref[...] = lax.dot_general(h, wep_ref[...], (((1,), (1,)), ((), ())),
                                  preferred_element_type=jnp.float32)


def _tc1(x, Wp, Wepair):
    return pl.pallas_call(
        _tc1_body,
        grid=(_GRID,),
        in_specs=[
            pl.BlockSpec((_BLK, H), lambda i: (i, 0)),
            pl.BlockSpec((H, H), lambda i: (0, 0)),
            pl.BlockSpec((2, H), lambda i: (0, 0)),
        ],
        out_specs=[
            pl.BlockSpec((_BLK, H), lambda i: (i, 0)),
            pl.BlockSpec((_BLK, 2), lambda i: (i, 0)),
        ],
        out_shape=[
            jax.ShapeDtypeStruct((NPAD, H), jnp.float32),
            jax.ShapeDtypeStruct((NPAD, 2), jnp.float32),
        ],
    )(x, Wp, Wepair)


def _tc2_body(agg_ref, den_ref, x_ref, g_ref, b_ref, o_ref):
    a = agg_ref[0] + agg_ref[1]
    d = den_ref[0] + den_ref[1] + 1e-6
    y = a / d[:, None] + x_ref[...]
    mu = jnp.mean(y, axis=-1, keepdims=True)
    var = jnp.mean((y - mu) ** 2, axis=-1, keepdims=True)
    y = (y - mu) / jnp.sqrt(var + 1e-5) * g_ref[...] + b_ref[...]
    o_ref[...] = 0.5 * y * (1.0 + lax.erf(y / math.sqrt(2.0)))


def _tc2(agg, den, x, g, b):
    return pl.pallas_call(
        _tc2_body,
        grid=(_GRID,),
        in_specs=[
            pl.BlockSpec((NC, _BLK, H), lambda i: (0, i, 0)),
            pl.BlockSpec((NC, _BLK), lambda i: (0, i)),
            pl.BlockSpec((_BLK, H), lambda i: (i, 0)),
            pl.BlockSpec((1, H), lambda i: (0, 0)),
            pl.BlockSpec((1, H), lambda i: (0, 0)),
        ],
        out_specs=pl.BlockSpec((_BLK, H), lambda i: (i, 0)),
        out_shape=jax.ShapeDtypeStruct((NPAD, H), jnp.float32),
    )(agg, den, x, g, b)


def _tc3_body(x_ref, wq_ref, bq_ref, swr_ref, mask_ref, vec_ref, w_ref):
    x = x_ref[...]
    proj = jnp.tanh(jnp.dot(x, wq_ref[...],
                            preferred_element_type=jnp.float32) + bq_ref[...])
    sc = lax.dot_general(proj, swr_ref[...], (((1,), (1,)), ((), ())),
                         preferred_element_type=jnp.float32)[:, 0]
    sc = jnp.where(mask_ref[...] > 0.0, sc, -jnp.inf)
    m = jnp.max(sc)
    e = jnp.exp(sc - m)
    wgt = e / jnp.sum(e)
    wgt = jnp.where(jnp.isnan(wgt), 0.0, wgt)
    w_ref[...] = wgt
    gv = lax.dot_general(wgt, x, (((0,), (0,)), ((), ())),
                         preferred_element_type=jnp.float32)
    vec_ref[...] = gv[None, :]


def _tc3(x, Wq, bq, swr, mask):
    return pl.pallas_call(
        _tc3_body,
        out_shape=[
            jax.ShapeDtypeStruct((1, H), jnp.float32),
            jax.ShapeDtypeStruct((NPAD,), jnp.float32),
        ],
    )(x, Wq, bq, swr, mask)


def _gat_layer(x, src, dst, srcq, dstq, Wp, Wepair, g, b):
    h, s2 = _tc1(x, Wp, Wepair)
    scores, maxes = _k1(s2[:, 0], s2[:, 1], src, dst)
    agg, den = _k2(scores, maxes, srcq, dstq, h)
    return _tc2(agg, den, x, g, b)


def kernel(node_ids, edge_index, emb, W1p, W1e, g1, b1, W2p, W2e, g2, b2,
           Wq, bq, sw):
    f32 = jnp.float32
    ids = jnp.concatenate([node_ids.astype(jnp.int32),
                           jnp.zeros((NPAD - N,), jnp.int32)])
    idsq = ids.reshape(NW, 5, 64)
    src = edge_index[0].astype(jnp.int32)
    dst = edge_index[1].astype(jnp.int32)
    srcq = src.reshape(NW, NCHUNK, CH)
    dstq = dst.reshape(NW, NCHUNK, CH)
    mask = (ids != 0).astype(f32)

    x = _k0(emb, idsq)
    We1 = W1e[:, 0].reshape(2, H)
    We2 = W2e[:, 0].reshape(2, H)
    x = _gat_layer(x, src, dst, srcq, dstq, W1p, We1,
                   g1.reshape(1, H), b1.reshape(1, H))
    x = _gat_layer(x, src, dst, srcq, dstq, W2p, We2,
                   g2.reshape(1, H), b2.reshape(1, H))

    vec, wts = _tc3(x, Wq, bq.reshape(1, H), sw[:, 0].reshape(1, H), mask)
    return (vec, wts[:N], jnp.ones((1,), f32))


# speedup vs baseline: 11.9143x; 1.4910x over previous
import functools
import math

import jax
import jax.numpy as jnp
from jax import lax
from jax.experimental import pallas as pl
from jax.experimental.pallas import tpu as pltpu
from jax.experimental.pallas import tpu_sc as plsc

N = 10000
E = 320000
H = 128
NPAD = 10240
NC, NS = 2, 16
NW = NC * NS
EPW = E // NW
CH = 80
NCHUNK = EPW // CH
BPW = NPAD // NW
RPS = NPAD // NS

_MESH = dict(core_axis_name="c", subcore_axis_name="s")


def _wid():
    return lax.axis_index("s") * NC + lax.axis_index("c")


def _k0_body(emb_hbm, idsq_hbm, x_hbm, idx_v, rows_v, sem):
    w = _wid()
    pltpu.sync_copy(idsq_hbm.at[w], idx_v)
    for j in range(5):
        pltpu.async_copy(emb_hbm.at[idx_v.at[j]],
                         rows_v.at[pl.ds(64 * j, 64)], sem).wait()
    pltpu.sync_copy(rows_v, x_hbm.at[pl.ds(w * BPW, BPW)])


def _k0(emb, idsq):
    return pl.kernel(
        _k0_body,
        out_type=jax.ShapeDtypeStruct((NPAD, H), jnp.float32),
        mesh=plsc.VectorSubcoreMesh(**_MESH),
        compiler_params=pltpu.CompilerParams(needs_layout_passes=False),
        scratch_types=[
            pltpu.VMEM((5, 64), jnp.int32),
            pltpu.VMEM((BPW, H), jnp.float32),
            pltpu.SemaphoreType.DMA,
        ],
    )(emb, idsq)


def _k1_body(ssrc_hbm, sdst_hbm, src_hbm, dst_hbm, scores_hbm, maxes_hbm,
             ssrc_v, sdst_v, src_v, dst_v, sc_v, mx_v, sem):
    w = _wid()
    pltpu.sync_copy(ssrc_hbm, ssrc_v)
    pltpu.sync_copy(sdst_hbm, sdst_v)
    pltpu.sync_copy(src_hbm.at[pl.ds(w * EPW, EPW)], src_v)
    pltpu.sync_copy(dst_hbm.at[pl.ds(w * EPW, EPW)], dst_v)

    def body(k, mx):
        o = pl.multiple_of(16 * k, 16)
        vs = plsc.load_gather(ssrc_v, [src_v[pl.ds(o, 16)]])
        vd = plsc.load_gather(sdst_v, [dst_v[pl.ds(o, 16)]])
        s = vs + vd
        s = jnp.where(s >= 0.0, s, 0.2 * s)
        sc_v[pl.ds(o, 16)] = s
        return jnp.maximum(mx, s)

    mx = lax.fori_loop(0, EPW // 16, body,
                       jnp.full((16,), -3.0e38, jnp.float32))
    mx_v[...] = mx
    pltpu.sync_copy(sc_v, scores_hbm.at[pl.ds(w * EPW, EPW)])
    pltpu.sync_copy(mx_v, maxes_hbm.at[pl.ds(16 * w, 16)])


def _k1(ssrc, sdst, src, dst):
    return pl.kernel(
        _k1_body,
        out_type=(jax.ShapeDtypeStruct((E,), jnp.float32),
                  jax.ShapeDtypeStruct((16 * NW,), jnp.float32)),
        mesh=plsc.VectorSubcoreMesh(**_MESH),
        compiler_params=pltpu.CompilerParams(needs_layout_passes=False),
        scratch_types=[
            pltpu.VMEM((NPAD,), jnp.float32),
            pltpu.VMEM((NPAD,), jnp.float32),
            pltpu.VMEM((EPW,), jnp.int32),
            pltpu.VMEM((EPW,), jnp.int32),
            pltpu.VMEM((EPW,), jnp.float32),
            pltpu.VMEM((16,), jnp.float32),
            pltpu.SemaphoreType.DMA,
        ],
    )(ssrc, sdst, src, dst)


NB = 5
CPB = NCHUNK // NB
NPAIR = CPB // 2


def _k2_body(scores_hbm, maxes_hbm, srcq_hbm, dstq_hbm, h_hbm,
             agg_out, den_out,
             agg_sh, den_sh,
             idxs_v, idxd_v, sc_v, rows0_v, rows1_v, maxm_v, zden_v,
             semg0, semg1, sems0, sems1, semd):
    c = lax.axis_index("c")
    s = lax.axis_index("s")
    w = s * NC + c

    z16 = jnp.zeros((16,), jnp.float32)
    for i in range(CH):
        for j in range(H // 16):
            rows0_v[i, pl.ds(16 * j, 16)] = z16
    for k in range(RPS // 16):
        zden_v[pl.ds(16 * k, 16)] = z16
    for k in range(RPS // CH):
        pltpu.sync_copy(rows0_v, agg_sh.at[pl.ds(RPS * s + CH * k, CH)])
    pltpu.sync_copy(zden_v, den_sh.at[pl.ds(RPS * s, RPS)])
    plsc.subcore_barrier()

    pltpu.sync_copy(maxes_hbm, maxm_v)
    pltpu.sync_copy(scores_hbm.at[pl.ds(w * EPW, EPW)], sc_v)

    def maxbody(k, m):
        o = pl.multiple_of(16 * k, 16)
        return jnp.maximum(m, maxm_v[pl.ds(o, 16)])

    m16 = lax.fori_loop(0, NW, maxbody, jnp.full((16,), -3.0e38, jnp.float32))
    M = jnp.max(m16)

    def expbody(k, _):
        o = pl.multiple_of(16 * k, 16)
        sc_v[pl.ds(o, 16)] = jnp.exp(sc_v[pl.ds(o, 16)] - M)
        return 0

    lax.fori_loop(0, EPW // 16, expbody, 0)

    def fire_gather(j, buf, semg):
        pltpu.async_copy(h_hbm.at[idxs_v.at[j]], buf, semg)

    def wait_gather(buf, semg):
        pltpu.make_async_copy(h_hbm.at[idxs_v.at[0]], buf, semg).wait()

    def fire_scatter(j, buf, sems):
        pltpu.async_copy(buf, agg_sh.at[idxd_v.at[j]], sems, add=True)

    def wait_scatter(buf, sems):
        pltpu.make_async_copy(rows0_v, agg_sh.at[idxd_v.at[0]], sems).wait()

    def fire_den(bi, j):
        ci = bi * CPB + j
        pltpu.async_copy(sc_v.at[pl.ds(ci * CH, CH)],
                         den_sh.at[idxd_v.at[j]], semd, add=True)

    def scale(buf, bi, j):
        base = (bi * CPB + j) * CH

        def edge(i, _):
            a = plsc.load_gather(sc_v, [lax.broadcast(base + i, (16,))])
            for jj in range(H // 16):
                buf[i, pl.ds(16 * jj, 16)] = buf[i, pl.ds(16 * jj, 16)] * a
            return 0

        lax.fori_loop(0, CH, edge, 0)

    for bi in range(NB):
        pltpu.sync_copy(srcq_hbm.at[w].at[bi], idxs_v)
        pltpu.sync_copy(dstq_hbm.at[w].at[bi], idxd_v)
        fire_gather(0, rows0_v, semg0)

        def pair(g, _):
            wait_gather(rows0_v, semg0)
            scale(rows0_v, bi, 2 * g)

            @pl.when(g > 0)
            def _():
                wait_scatter(rows1_v, sems1)

            fire_gather(2 * g + 1, rows1_v, semg1)
            fire_scatter(2 * g, rows0_v, sems0)
            fire_den(bi, 2 * g)
            wait_gather(rows1_v, semg1)
            scale(rows1_v, bi, 2 * g + 1)
            wait_scatter(rows0_v, sems0)
            fire_gather(2 * g + 2, rows0_v, semg0)
            fire_scatter(2 * g + 1, rows1_v, sems1)
            fire_den(bi, 2 * g + 1)
            return 0

        lax.fori_loop(0, NPAIR, pair, 0)
        wait_scatter(rows1_v, sems1)
        wait_gather(rows0_v, semg0)
        scale(rows0_v, bi, CPB - 1)
        fire_scatter(CPB - 1, rows0_v, sems0)
        fire_den(bi, CPB - 1)
        wait_scatter(rows0_v, sems0)

        def draind(j, _):
            pltpu.make_async_copy(sc_v.at[pl.ds(0, CH)],
                                  den_sh.at[idxd_v.at[0]], semd).wait()
            return 0

        lax.fori_loop(0, CPB, draind, 0)

    plsc.subcore_barrier()

    for k in range(RPS // CH):
        pltpu.sync_copy(agg_sh.at[pl.ds(RPS * s + CH * k, CH)], rows0_v)
        pltpu.sync_copy(rows0_v, agg_out.at[c].at[pl.ds(RPS * s + CH * k, CH)])
    pltpu.sync_copy(den_sh.at[pl.ds(RPS * s, RPS)], zden_v)
    pltpu.sync_copy(zden_v, den_out.at[c].at[pl.ds(RPS * s, RPS)])


def _k2(scores, maxes, srcq, dstq, h):
    return pl.kernel(
        _k2_body,
        out_type=(jax.ShapeDtypeStruct((NC, NPAD, H), jnp.float32),
                  jax.ShapeDtypeStruct((NC, NPAD), jnp.float32)),
        mesh=plsc.VectorSubcoreMesh(**_MESH),
        compiler_params=pltpu.CompilerParams(needs_layout_passes=False),
        scratch_types=[
            pltpu.VMEM_SHARED((NPAD, H), jnp.float32),
            pltpu.VMEM_SHARED((NPAD,), jnp.float32),
            pltpu.VMEM((CPB, CH), jnp.int32),
            pltpu.VMEM((CPB, CH), jnp.int32),
            pltpu.VMEM((EPW,), jnp.float32),
            pltpu.VMEM((CH, H), jnp.float32),
            pltpu.VMEM((CH, H), jnp.float32),
            pltpu.VMEM((16 * NW,), jnp.float32),
            pltpu.VMEM((RPS,), jnp.float32),
            pltpu.SemaphoreType.DMA,
            pltpu.SemaphoreType.DMA,
            pltpu.SemaphoreType.DMA,
            pltpu.SemaphoreType.DMA,
            pltpu.SemaphoreType.DMA,
        ],
    )(scores, maxes, srcq, dstq, h)


_BLK = 1024
_GRID = NPAD // _BLK


def _tc1_body(x_ref, wp_ref, wep_ref, h_ref, s2_ref):
    h = jnp.dot(x_ref[...], wp_ref[...], preferred_element_type=jnp.float32)
    h_ref[...] = h
    s2_ref[...] = lax.dot_general(h, wep_ref[...], (((1,), (1,)), ((), ())),
                                  preferred_element_type=jnp.float32)


def _tc1(x, Wp, Wepair):
    return pl.pallas_call(
        _tc1_body,
        grid=(_GRID,),
        in_specs=[
            pl.BlockSpec((_BLK, H), lambda i: (i, 0)),
            pl.BlockSpec((H, H), lambda i: (0, 0)),
            pl.BlockSpec((2, H), lambda i: (0, 0)),
        ],
        out_specs=[
            pl.BlockSpec((_BLK, H), lambda i: (i, 0)),
            pl.BlockSpec((_BLK, 2), lambda i: (i, 0)),
        ],
        out_shape=[
            jax.ShapeDtypeStruct((NPAD, H), jnp.float32),
            jax.ShapeDtypeStruct((NPAD, 2), jnp.float32),
        ],
    )(x, Wp, Wepair)


def _tc2_body(agg_ref, den_ref, x_ref, g_ref, b_ref, o_ref):
    a = agg_ref[0] + agg_ref[1]
    d = den_ref[0] + den_ref[1] + 1e-6
    y = a / d[:, None] + x_ref[...]
    mu = jnp.mean(y, axis=-1, keepdims=True)
    var = jnp.mean((y - mu) ** 2, axis=-1, keepdims=True)
    y = (y - mu) / jnp.sqrt(var + 1e-5) * g_ref[...] + b_ref[...]
    o_ref[...] = 0.5 * y * (1.0 + lax.erf(y / math.sqrt(2.0)))


def _tc2(agg, den, x, g, b):
    return pl.pallas_call(
        _tc2_body,
        grid=(_GRID,),
        in_specs=[
            pl.BlockSpec((NC, _BLK, H), lambda i: (0, i, 0)),
            pl.BlockSpec((NC, _BLK), lambda i: (0, i)),
            pl.BlockSpec((_BLK, H), lambda i: (i, 0)),
            pl.BlockSpec((1, H), lambda i: (0, 0)),
            pl.BlockSpec((1, H), lambda i: (0, 0)),
        ],
        out_specs=pl.BlockSpec((_BLK, H), lambda i: (i, 0)),
        out_shape=jax.ShapeDtypeStruct((NPAD, H), jnp.float32),
    )(agg, den, x, g, b)


def _tc3_body(x_ref, wq_ref, bq_ref, swr_ref, mask_ref, vec_ref, w_ref):
    x = x_ref[...]
    proj = jnp.tanh(jnp.dot(x, wq_ref[...],
                            preferred_element_type=jnp.float32) + bq_ref[...])
    sc = lax.dot_general(proj, swr_ref[...], (((1,), (1,)), ((), ())),
                         preferred_element_type=jnp.float32)[:, 0]
    sc = jnp.where(mask_ref[...] > 0.0, sc, -jnp.inf)
    m = jnp.max(sc)
    e = jnp.exp(sc - m)
    wgt = e / jnp.sum(e)
    wgt = jnp.where(jnp.isnan(wgt), 0.0, wgt)
    w_ref[...] = wgt
    gv = lax.dot_general(wgt, x, (((0,), (0,)), ((), ())),
                         preferred_element_type=jnp.float32)
    vec_ref[...] = gv[None, :]


def _tc3(x, Wq, bq, swr, mask):
    return pl.pallas_call(
        _tc3_body,
        out_shape=[
            jax.ShapeDtypeStruct((1, H), jnp.float32),
            jax.ShapeDtypeStruct((NPAD,), jnp.float32),
        ],
    )(x, Wq, bq, swr, mask)


def _gat_layer(x, src, dst, srcq, dstq, Wp, Wepair, g, b):
    h, s2 = _tc1(x, Wp, Wepair)
    scores, maxes = _k1(s2[:, 0], s2[:, 1], src, dst)
    agg, den = _k2(scores, maxes, srcq, dstq, h)
    return _tc2(agg, den, x, g, b)


def kernel(node_ids, edge_index, emb, W1p, W1e, g1, b1, W2p, W2e, g2, b2,
           Wq, bq, sw):
    f32 = jnp.float32
    ids = jnp.concatenate([node_ids.astype(jnp.int32),
                           jnp.zeros((NPAD - N,), jnp.int32)])
    idsq = ids.reshape(NW, 5, 64)
    src = edge_index[0].astype(jnp.int32)
    dst = edge_index[1].astype(jnp.int32)
    srcq = src.reshape(NW, NB, CPB, CH)
    dstq = dst.reshape(NW, NB, CPB, CH)
    mask = (ids != 0).astype(f32)

    x = _k0(emb, idsq)
    We1 = W1e[:, 0].reshape(2, H)
    We2 = W2e[:, 0].reshape(2, H)
    x = _gat_layer(x, src, dst, srcq, dstq, W1p, We1,
                   g1.reshape(1, H), b1.reshape(1, H))
    x = _gat_layer(x, src, dst, srcq, dstq, W2p, We2,
                   g2.reshape(1, H), b2.reshape(1, H))

    vec, wts = _tc3(x, Wq, bq.reshape(1, H), sw[:, 0].reshape(1, H), mask)
    return (vec, wts[:N], jnp.ones((1,), f32))


# speedup vs baseline: 13.7972x; 1.1580x over previous
import functools
import math

import jax
import jax.numpy as jnp
from jax import lax
from jax.experimental import pallas as pl
from jax.experimental.pallas import tpu as pltpu
from jax.experimental.pallas import tpu_sc as plsc

N = 10000
E = 320000
H = 128
NPAD = 10240
NC, NS = 2, 16
NW = NC * NS
EPW = E // NW
CH = 80
NCHUNK = EPW // CH
BPW = NPAD // NW
RPS = NPAD // NS

_MESH = dict(core_axis_name="c", subcore_axis_name="s")


def _wid():
    return lax.axis_index("s") * NC + lax.axis_index("c")


def _k0_body(emb_hbm, idsq_hbm, x_hbm, idx_v, rows_v, sem):
    w = _wid()
    pltpu.sync_copy(idsq_hbm.at[w], idx_v)
    for j in range(5):
        pltpu.async_copy(emb_hbm.at[idx_v.at[j]],
                         rows_v.at[pl.ds(64 * j, 64)], sem).wait()
    pltpu.sync_copy(rows_v, x_hbm.at[pl.ds(w * BPW, BPW)])


def _k0(emb, idsq):
    return pl.kernel(
        _k0_body,
        out_type=jax.ShapeDtypeStruct((NPAD, H), jnp.float32),
        mesh=plsc.VectorSubcoreMesh(**_MESH),
        compiler_params=pltpu.CompilerParams(needs_layout_passes=False),
        scratch_types=[
            pltpu.VMEM((5, 64), jnp.int32),
            pltpu.VMEM((BPW, H), jnp.float32),
            pltpu.SemaphoreType.DMA,
        ],
    )(emb, idsq)


def _k1_body(ssrc_hbm, sdst_hbm, src_hbm, dst_hbm, scores_hbm, maxes_hbm,
             ssrc_v, sdst_v, src_v, dst_v, sc_v, mx_v, sem):
    w = _wid()
    pltpu.sync_copy(ssrc_hbm, ssrc_v)
    pltpu.sync_copy(sdst_hbm, sdst_v)
    pltpu.sync_copy(src_hbm.at[pl.ds(w * EPW, EPW)], src_v)
    pltpu.sync_copy(dst_hbm.at[pl.ds(w * EPW, EPW)], dst_v)

    def body(k, mx):
        o = pl.multiple_of(16 * k, 16)
        vs = plsc.load_gather(ssrc_v, [src_v[pl.ds(o, 16)]])
        vd = plsc.load_gather(sdst_v, [dst_v[pl.ds(o, 16)]])
        s = vs + vd
        s = jnp.where(s >= 0.0, s, 0.2 * s)
        sc_v[pl.ds(o, 16)] = s
        return jnp.maximum(mx, s)

    mx = lax.fori_loop(0, EPW // 16, body,
                       jnp.full((16,), -3.0e38, jnp.float32))
    mx_v[...] = mx
    pltpu.sync_copy(sc_v, scores_hbm.at[pl.ds(w * EPW, EPW)])
    pltpu.sync_copy(mx_v, maxes_hbm.at[pl.ds(16 * w, 16)])


def _k1(ssrc, sdst, src, dst):
    return pl.kernel(
        _k1_body,
        out_type=(jax.ShapeDtypeStruct((E,), jnp.float32),
                  jax.ShapeDtypeStruct((16 * NW,), jnp.float32)),
        mesh=plsc.VectorSubcoreMesh(**_MESH),
        compiler_params=pltpu.CompilerParams(needs_layout_passes=False),
        scratch_types=[
            pltpu.VMEM((NPAD,), jnp.float32),
            pltpu.VMEM((NPAD,), jnp.float32),
            pltpu.VMEM((EPW,), jnp.int32),
            pltpu.VMEM((EPW,), jnp.int32),
            pltpu.VMEM((EPW,), jnp.float32),
            pltpu.VMEM((16,), jnp.float32),
            pltpu.SemaphoreType.DMA,
        ],
    )(ssrc, sdst, src, dst)


NB = 5
CPB = NCHUNK // NB
NPAIR = CPB // 2


def _k2_body(scores_hbm, maxes_hbm, srcq_hbm, dstq_hbm, h_hbm,
             agg_out, den_out,
             agg_sh, den_sh,
             idxs_v, idxd_v, sc_v, rows0_v, rows1_v, maxm_v, zden_v,
             semg0, semg1, sems0, sems1, semd):
    c = lax.axis_index("c")
    s = lax.axis_index("s")
    w = s * NC + c

    z16 = jnp.zeros((16,), jnp.float32)
    for i in range(CH):
        for j in range(H // 16):
            rows0_v[i, pl.ds(16 * j, 16)] = z16
    for k in range(RPS // 16):
        zden_v[pl.ds(16 * k, 16)] = z16
    for k in range(RPS // CH):
        pltpu.sync_copy(rows0_v, agg_sh.at[pl.ds(RPS * s + CH * k, CH)])
    pltpu.sync_copy(zden_v, den_sh.at[pl.ds(RPS * s, RPS)])
    plsc.subcore_barrier()

    pltpu.sync_copy(maxes_hbm, maxm_v)
    pltpu.sync_copy(scores_hbm.at[pl.ds(w * EPW, EPW)], sc_v)

    def maxbody(k, m):
        o = pl.multiple_of(16 * k, 16)
        return jnp.maximum(m, maxm_v[pl.ds(o, 16)])

    m16 = lax.fori_loop(0, NW, maxbody, jnp.full((16,), -3.0e38, jnp.float32))
    M = jnp.max(m16)

    def expbody(k, _):
        o = pl.multiple_of(16 * k, 16)
        sc_v[pl.ds(o, 16)] = jnp.exp(sc_v[pl.ds(o, 16)] - M)
        return 0

    lax.fori_loop(0, EPW // 16, expbody, 0)

    def fire_gather(j, buf, semg):
        pltpu.async_copy(h_hbm.at[idxs_v.at[j]], buf, semg)

    def wait_gather(buf, semg):
        pltpu.make_async_copy(h_hbm.at[idxs_v.at[0]], buf, semg).wait()

    def fire_scatter(j, buf, sems):
        pltpu.async_copy(buf, agg_sh.at[idxd_v.at[j]], sems, add=True)

    def wait_scatter(buf, sems):
        pltpu.make_async_copy(rows0_v, agg_sh.at[idxd_v.at[0]], sems).wait()

    def fire_den(bi, j):
        ci = bi * CPB + j
        pltpu.async_copy(sc_v.at[pl.ds(ci * CH, CH)],
                         den_sh.at[idxd_v.at[j]], semd, add=True)

    def scale(buf, bi, j):
        base = (bi * CPB + j) * CH

        def sub(k, _):
            o = pl.multiple_of(base + 16 * k, 16)
            av = sc_v[pl.ds(o, 16)]
            r0 = 16 * k
            for i in range(16):
                a = lax.broadcast(av[i], (16,))
                for jj in range(H // 16):
                    buf[r0 + i, pl.ds(16 * jj, 16)] = (
                        buf[r0 + i, pl.ds(16 * jj, 16)] * a)
            return 0

        lax.fori_loop(0, CH // 16, sub, 0)

    for bi in range(NB):
        pltpu.sync_copy(srcq_hbm.at[w].at[bi], idxs_v)
        pltpu.sync_copy(dstq_hbm.at[w].at[bi], idxd_v)
        fire_gather(0, rows0_v, semg0)

        def pair(g, _):
            wait_gather(rows0_v, semg0)
            scale(rows0_v, bi, 2 * g)

            @pl.when(g > 0)
            def _():
                wait_scatter(rows1_v, sems1)

            fire_gather(2 * g + 1, rows1_v, semg1)
            fire_scatter(2 * g, rows0_v, sems0)
            fire_den(bi, 2 * g)
            wait_gather(rows1_v, semg1)
            scale(rows1_v, bi, 2 * g + 1)
            wait_scatter(rows0_v, sems0)
            fire_gather(2 * g + 2, rows0_v, semg0)
            fire_scatter(2 * g + 1, rows1_v, sems1)
            fire_den(bi, 2 * g + 1)
            return 0

        lax.fori_loop(0, NPAIR, pair, 0)
        wait_scatter(rows1_v, sems1)
        wait_gather(rows0_v, semg0)
        scale(rows0_v, bi, CPB - 1)
        fire_scatter(CPB - 1, rows0_v, sems0)
        fire_den(bi, CPB - 1)
        wait_scatter(rows0_v, sems0)

        def draind(j, _):
            pltpu.make_async_copy(sc_v.at[pl.ds(0, CH)],
                                  den_sh.at[idxd_v.at[0]], semd).wait()
            return 0

        lax.fori_loop(0, CPB, draind, 0)

    plsc.subcore_barrier()

    for k in range(RPS // CH):
        pltpu.sync_copy(agg_sh.at[pl.ds(RPS * s + CH * k, CH)], rows0_v)
        pltpu.sync_copy(rows0_v, agg_out.at[c].at[pl.ds(RPS * s + CH * k, CH)])
    pltpu.sync_copy(den_sh.at[pl.ds(RPS * s, RPS)], zden_v)
    pltpu.sync_copy(zden_v, den_out.at[c].at[pl.ds(RPS * s, RPS)])


def _k2(scores, maxes, srcq, dstq, h):
    return pl.kernel(
        _k2_body,
        out_type=(jax.ShapeDtypeStruct((NC, NPAD, H), jnp.float32),
                  jax.ShapeDtypeStruct((NC, NPAD), jnp.float32)),
        mesh=plsc.VectorSubcoreMesh(**_MESH),
        compiler_params=pltpu.CompilerParams(needs_layout_passes=False),
        scratch_types=[
            pltpu.VMEM_SHARED((NPAD, H), jnp.float32),
            pltpu.VMEM_SHARED((NPAD,), jnp.float32),
            pltpu.VMEM((CPB, CH), jnp.int32),
            pltpu.VMEM((CPB, CH), jnp.int32),
            pltpu.VMEM((EPW,), jnp.float32),
            pltpu.VMEM((CH, H), jnp.float32),
            pltpu.VMEM((CH, H), jnp.float32),
            pltpu.VMEM((16 * NW,), jnp.float32),
            pltpu.VMEM((RPS,), jnp.float32),
            pltpu.SemaphoreType.DMA,
            pltpu.SemaphoreType.DMA,
            pltpu.SemaphoreType.DMA,
            pltpu.SemaphoreType.DMA,
            pltpu.SemaphoreType.DMA,
        ],
    )(scores, maxes, srcq, dstq, h)


_BLK = 1024
_GRID = NPAD // _BLK


def _tc1_body(x_ref, wp_ref, wep_ref, h_ref, s2_ref):
    h = jnp.dot(x_ref[...], wp_ref[...], preferred_element_type=jnp.float32)
    h_ref[...] = h
    s2_ref[...] = lax.dot_general(h, wep_ref[...], (((1,), (1,)), ((), ())),
                                  preferred_element_type=jnp.float32)


def _tc1(x, Wp, Wepair):
    return pl.pallas_call(
        _tc1_body,
        grid=(_GRID,),
        in_specs=[
            pl.BlockSpec((_BLK, H), lambda i: (i, 0)),
            pl.BlockSpec((H, H), lambda i: (0, 0)),
            pl.BlockSpec((2, H), lambda i: (0, 0)),
        ],
        out_specs=[
            pl.BlockSpec((_BLK, H), lambda i: (i, 0)),
            pl.BlockSpec((_BLK, 2), lambda i: (i, 0)),
        ],
        out_shape=[
            jax.ShapeDtypeStruct((NPAD, H), jnp.float32),
            jax.ShapeDtypeStruct((NPAD, 2), jnp.float32),
        ],
    )(x, Wp, Wepair)


def _tc2_body(agg_ref, den_ref, x_ref, g_ref, b_ref, o_ref):
    a = agg_ref[0] + agg_ref[1]
    d = den_ref[0] + den_ref[1] + 1e-6
    y = a / d[:, None] + x_ref[...]
    mu = jnp.mean(y, axis=-1, keepdims=True)
    var = jnp.mean((y - mu) ** 2, axis=-1, keepdims=True)
    y = (y - mu) / jnp.sqrt(var + 1e-5) * g_ref[...] + b_ref[...]
    o_ref[...] = 0.5 * y * (1.0 + lax.erf(y / math.sqrt(2.0)))


def _tc2(agg, den, x, g, b):
    return pl.pallas_call(
        _tc2_body,
        grid=(_GRID,),
        in_specs=[
            pl.BlockSpec((NC, _BLK, H), lambda i: (0, i, 0)),
            pl.BlockSpec((NC, _BLK), lambda i: (0, i)),
            pl.BlockSpec((_BLK, H), lambda i: (i, 0)),
            pl.BlockSpec((1, H), lambda i: (0, 0)),
            pl.BlockSpec((1, H), lambda i: (0, 0)),
        ],
        out_specs=pl.BlockSpec((_BLK, H), lambda i: (i, 0)),
        out_shape=jax.ShapeDtypeStruct((NPAD, H), jnp.float32),
    )(agg, den, x, g, b)


def _tc3_body(x_ref, wq_ref, bq_ref, swr_ref, mask_ref, vec_ref, w_ref):
    x = x_ref[...]
    proj = jnp.tanh(jnp.dot(x, wq_ref[...],
                            preferred_element_type=jnp.float32) + bq_ref[...])
    sc = lax.dot_general(proj, swr_ref[...], (((1,), (1,)), ((), ())),
                         preferred_element_type=jnp.float32)[:, 0]
    sc = jnp.where(mask_ref[...] > 0.0, sc, -jnp.inf)
    m = jnp.max(sc)
    e = jnp.exp(sc - m)
    wgt = e / jnp.sum(e)
    wgt = jnp.where(jnp.isnan(wgt), 0.0, wgt)
    w_ref[...] = wgt
    gv = lax.dot_general(wgt, x, (((0,), (0,)), ((), ())),
                         preferred_element_type=jnp.float32)
    vec_ref[...] = gv[None, :]


def _tc3(x, Wq, bq, swr, mask):
    return pl.pallas_call(
        _tc3_body,
        out_shape=[
            jax.ShapeDtypeStruct((1, H), jnp.float32),
            jax.ShapeDtypeStruct((NPAD,), jnp.float32),
        ],
    )(x, Wq, bq, swr, mask)


def _gat_layer(x, src, dst, srcq, dstq, Wp, Wepair, g, b):
    h, s2 = _tc1(x, Wp, Wepair)
    scores, maxes = _k1(s2[:, 0], s2[:, 1], src, dst)
    agg, den = _k2(scores, maxes, srcq, dstq, h)
    return _tc2(agg, den, x, g, b)


def kernel(node_ids, edge_index, emb, W1p, W1e, g1, b1, W2p, W2e, g2, b2,
           Wq, bq, sw):
    f32 = jnp.float32
    ids = jnp.concatenate([node_ids.astype(jnp.int32),
                           jnp.zeros((NPAD - N,), jnp.int32)])
    idsq = ids.reshape(NW, 5, 64)
    src = edge_index[0].astype(jnp.int32)
    dst = edge_index[1].astype(jnp.int32)
    srcq = src.reshape(NW, NB, CPB, CH)
    dstq = dst.reshape(NW, NB, CPB, CH)
    mask = (ids != 0).astype(f32)

    x = _k0(emb, idsq)
    We1 = W1e[:, 0].reshape(2, H)
    We2 = W2e[:, 0].reshape(2, H)
    x = _gat_layer(x, src, dst, srcq, dstq, W1p, We1,
                   g1.reshape(1, H), b1.reshape(1, H))
    x = _gat_layer(x, src, dst, srcq, dstq, W2p, We2,
                   g2.reshape(1, H), b2.reshape(1, H))

    vec, wts = _tc3(x, Wq, bq.reshape(1, H), sw[:, 0].reshape(1, H), mask)
    return (vec, wts[:N], jnp.ones((1,), f32))


# speedup vs baseline: 14.9212x; 1.0815x over previous
import functools
import math

import jax
import jax.numpy as jnp
from jax import lax
from jax.experimental import pallas as pl
from jax.experimental.pallas import tpu as pltpu
from jax.experimental.pallas import tpu_sc as plsc

N = 10000
E = 320000
H = 128
NPAD = 10240
NC, NS = 2, 16
NW = NC * NS
EPW = E // NW
CH = 80
NCHUNK = EPW // CH
BPW = NPAD // NW
RPS = NPAD // NS

_MESH = dict(core_axis_name="c", subcore_axis_name="s")


def _wid():
    return lax.axis_index("s") * NC + lax.axis_index("c")


def _k0_body(emb_hbm, idsq_hbm, x_hbm, idx_v, rows_v, sem):
    w = _wid()
    pltpu.sync_copy(idsq_hbm.at[w], idx_v)
    for j in range(5):
        pltpu.async_copy(emb_hbm.at[idx_v.at[j]],
                         rows_v.at[pl.ds(64 * j, 64)], sem).wait()
    pltpu.sync_copy(rows_v, x_hbm.at[pl.ds(w * BPW, BPW)])


def _k0(emb, idsq):
    return pl.kernel(
        _k0_body,
        out_type=jax.ShapeDtypeStruct((NPAD, H), jnp.float32),
        mesh=plsc.VectorSubcoreMesh(**_MESH),
        compiler_params=pltpu.CompilerParams(needs_layout_passes=False),
        scratch_types=[
            pltpu.VMEM((5, 64), jnp.int32),
            pltpu.VMEM((BPW, H), jnp.float32),
            pltpu.SemaphoreType.DMA,
        ],
    )(emb, idsq)


def _k1_body(ssrc_hbm, sdst_hbm, src_hbm, dst_hbm, scores_hbm, maxes_hbm,
             ssrc_v, sdst_v, src_v, dst_v, sc_v, mx_v, sem):
    w = _wid()
    pltpu.sync_copy(ssrc_hbm, ssrc_v)
    pltpu.sync_copy(sdst_hbm, sdst_v)
    pltpu.sync_copy(src_hbm.at[pl.ds(w * EPW, EPW)], src_v)
    pltpu.sync_copy(dst_hbm.at[pl.ds(w * EPW, EPW)], dst_v)

    def body(k, mx):
        o = pl.multiple_of(16 * k, 16)
        vs = plsc.load_gather(ssrc_v, [src_v[pl.ds(o, 16)]])
        vd = plsc.load_gather(sdst_v, [dst_v[pl.ds(o, 16)]])
        s = vs + vd
        s = jnp.where(s >= 0.0, s, 0.2 * s)
        sc_v[pl.ds(o, 16)] = s
        return jnp.maximum(mx, s)

    mx = lax.fori_loop(0, EPW // 16, body,
                       jnp.full((16,), -3.0e38, jnp.float32))
    mx_v[...] = mx
    pltpu.sync_copy(sc_v, scores_hbm.at[pl.ds(w * EPW, EPW)])
    pltpu.sync_copy(mx_v, maxes_hbm.at[pl.ds(16 * w, 16)])


def _k1(ssrc, sdst, src, dst):
    return pl.kernel(
        _k1_body,
        out_type=(jax.ShapeDtypeStruct((E,), jnp.float32),
                  jax.ShapeDtypeStruct((16 * NW,), jnp.float32)),
        mesh=plsc.VectorSubcoreMesh(**_MESH),
        compiler_params=pltpu.CompilerParams(needs_layout_passes=False),
        scratch_types=[
            pltpu.VMEM((NPAD,), jnp.float32),
            pltpu.VMEM((NPAD,), jnp.float32),
            pltpu.VMEM((EPW,), jnp.int32),
            pltpu.VMEM((EPW,), jnp.int32),
            pltpu.VMEM((EPW,), jnp.float32),
            pltpu.VMEM((16,), jnp.float32),
            pltpu.SemaphoreType.DMA,
        ],
    )(ssrc, sdst, src, dst)


NB = 5
CPB = NCHUNK // NB
NPAIR = CPB // 2


def _k2_body(scores_hbm, maxes_hbm, srcq_hbm, dstq_hbm, h_hbm,
             agg_out, den_out,
             agg_sh, den_sh,
             idxs_v, idxd_v, sc_v, rows0_v, rows1_v, maxm_v, zden_v,
             semg0, semg1, sems0, sems1, semd):
    c = lax.axis_index("c")
    s = lax.axis_index("s")
    w = s * NC + c

    z16 = jnp.zeros((16,), jnp.float32)
    for i in range(CH):
        for j in range(H // 16):
            rows0_v[i, pl.ds(16 * j, 16)] = z16
    for k in range(RPS // 16):
        zden_v[pl.ds(16 * k, 16)] = z16
    for k in range(RPS // CH):
        pltpu.sync_copy(rows0_v, agg_sh.at[pl.ds(RPS * s + CH * k, CH)])
    pltpu.sync_copy(zden_v, den_sh.at[pl.ds(RPS * s, RPS)])
    plsc.subcore_barrier()

    pltpu.sync_copy(maxes_hbm, maxm_v)
    pltpu.sync_copy(scores_hbm.at[pl.ds(w * EPW, EPW)], sc_v)

    def maxbody(k, m):
        o = pl.multiple_of(16 * k, 16)
        return jnp.maximum(m, maxm_v[pl.ds(o, 16)])

    m16 = lax.fori_loop(0, NW, maxbody, jnp.full((16,), -3.0e38, jnp.float32))
    M = jnp.max(m16)

    def expbody(k, _):
        o = pl.multiple_of(16 * k, 16)
        sc_v[pl.ds(o, 16)] = jnp.exp(sc_v[pl.ds(o, 16)] - M)
        return 0

    lax.fori_loop(0, EPW // 16, expbody, 0)

    def fire_gather(j, buf, semg):
        pltpu.async_copy(h_hbm.at[idxs_v.at[j]], buf, semg)

    def wait_gather(buf, semg):
        pltpu.make_async_copy(h_hbm.at[idxs_v.at[0]], buf, semg).wait()

    def fire_scatter(j, buf, sems):
        pltpu.async_copy(buf, agg_sh.at[idxd_v.at[j]], sems, add=True)

    def wait_scatter(buf, sems):
        pltpu.make_async_copy(rows0_v, agg_sh.at[idxd_v.at[0]], sems).wait()

    def fire_den(bi, j):
        ci = bi * CPB + j
        pltpu.async_copy(sc_v.at[pl.ds(ci * CH, CH)],
                         den_sh.at[idxd_v.at[j]], semd, add=True)

    def scale(buf, bi, j):
        base = (bi * CPB + j) * CH

        def sub(k, _):
            o = pl.multiple_of(base + 16 * k, 16)
            av = sc_v[pl.ds(o, 16)]
            r0 = 16 * k
            for i in range(16):
                a = lax.broadcast(av[i], (16,))
                for jj in range(H // 16):
                    buf[r0 + i, pl.ds(16 * jj, 16)] = (
                        buf[r0 + i, pl.ds(16 * jj, 16)] * a)
            return 0

        lax.fori_loop(0, CH // 16, sub, 0)

    for bi in range(NB):
        pltpu.sync_copy(srcq_hbm.at[w].at[bi], idxs_v)
        pltpu.sync_copy(dstq_hbm.at[w].at[bi], idxd_v)
        fire_gather(0, rows0_v, semg0)

        def pair(g, _):
            wait_gather(rows0_v, semg0)

            @pl.when(g > 0)
            def _():
                wait_scatter(rows1_v, sems1)

            fire_gather(2 * g + 1, rows1_v, semg1)
            scale(rows0_v, bi, 2 * g)
            fire_scatter(2 * g, rows0_v, sems0)
            fire_den(bi, 2 * g)
            wait_gather(rows1_v, semg1)
            scale(rows1_v, bi, 2 * g + 1)
            wait_scatter(rows0_v, sems0)
            fire_gather(2 * g + 2, rows0_v, semg0)
            fire_scatter(2 * g + 1, rows1_v, sems1)
            fire_den(bi, 2 * g + 1)
            return 0

        lax.fori_loop(0, NPAIR, pair, 0)
        wait_scatter(rows1_v, sems1)
        wait_gather(rows0_v, semg0)
        scale(rows0_v, bi, CPB - 1)
        fire_scatter(CPB - 1, rows0_v, sems0)
        fire_den(bi, CPB - 1)
        wait_scatter(rows0_v, sems0)

        def draind(j, _):
            pltpu.make_async_copy(sc_v.at[pl.ds(0, CH)],
                                  den_sh.at[idxd_v.at[0]], semd).wait()
            return 0

        lax.fori_loop(0, CPB, draind, 0)

    plsc.subcore_barrier()

    for k in range(RPS // CH):
        pltpu.sync_copy(agg_sh.at[pl.ds(RPS * s + CH * k, CH)], rows0_v)
        pltpu.sync_copy(rows0_v, agg_out.at[c].at[pl.ds(RPS * s + CH * k, CH)])
    pltpu.sync_copy(den_sh.at[pl.ds(RPS * s, RPS)], zden_v)
    pltpu.sync_copy(zden_v, den_out.at[c].at[pl.ds(RPS * s, RPS)])


def _k2(scores, maxes, srcq, dstq, h):
    return pl.kernel(
        _k2_body,
        out_type=(jax.ShapeDtypeStruct((NC, NPAD, H), jnp.float32),
                  jax.ShapeDtypeStruct((NC, NPAD), jnp.float32)),
        mesh=plsc.VectorSubcoreMesh(**_MESH),
        compiler_params=pltpu.CompilerParams(needs_layout_passes=False),
        scratch_types=[
            pltpu.VMEM_SHARED((NPAD, H), jnp.float32),
            pltpu.VMEM_SHARED((NPAD,), jnp.float32),
            pltpu.VMEM((CPB, CH), jnp.int32),
            pltpu.VMEM((CPB, CH), jnp.int32),
            pltpu.VMEM((EPW,), jnp.float32),
            pltpu.VMEM((CH, H), jnp.float32),
            pltpu.VMEM((CH, H), jnp.float32),
            pltpu.VMEM((16 * NW,), jnp.float32),
            pltpu.VMEM((RPS,), jnp.float32),
            pltpu.SemaphoreType.DMA,
            pltpu.SemaphoreType.DMA,
            pltpu.SemaphoreType.DMA,
            pltpu.SemaphoreType.DMA,
            pltpu.SemaphoreType.DMA,
        ],
    )(scores, maxes, srcq, dstq, h)


_BLK = 1024
_GRID = NPAD // _BLK


def _tc1_body(x_ref, wp_ref, wep_ref, h_ref, s2_ref):
    h = jnp.dot(x_ref[...], wp_ref[...], preferred_element_type=jnp.float32)
    h_ref[...] = h
    s2_ref[...] = lax.dot_general(h, wep_ref[...], (((1,), (1,)), ((), ())),
                                  preferred_element_type=jnp.float32)


def _tc1(x, Wp, Wepair):
    return pl.pallas_call(
        _tc1_body,
        grid=(_GRID,),
        in_specs=[
            pl.BlockSpec((_BLK, H), lambda i: (i, 0)),
            pl.BlockSpec((H, H), lambda i: (0, 0)),
            pl.BlockSpec((2, H), lambda i: (0, 0)),
        ],
        out_specs=[
            pl.BlockSpec((_BLK, H), lambda i: (i, 0)),
            pl.BlockSpec((_BLK, 2), lambda i: (i, 0)),
        ],
        out_shape=[
            jax.ShapeDtypeStruct((NPAD, H), jnp.float32),
            jax.ShapeDtypeStruct((NPAD, 2), jnp.float32),
        ],
    )(x, Wp, Wepair)


def _tc2_body(agg_ref, den_ref, x_ref, g_ref, b_ref, o_ref):
    a = agg_ref[0] + agg_ref[1]
    d = den_ref[0] + den_ref[1] + 1e-6
    y = a / d[:, None] + x_ref[...]
    mu = jnp.mean(y, axis=-1, keepdims=True)
    var = jnp.mean((y - mu) ** 2, axis=-1, keepdims=True)
    y = (y - mu) / jnp.sqrt(var + 1e-5) * g_ref[...] + b_ref[...]
    o_ref[...] = 0.5 * y * (1.0 + lax.erf(y / math.sqrt(2.0)))


def _tc2(agg, den, x, g, b):
    return pl.pallas_call(
        _tc2_body,
        grid=(_GRID,),
        in_specs=[
            pl.BlockSpec((NC, _BLK, H), lambda i: (0, i, 0)),
            pl.BlockSpec((NC, _BLK), lambda i: (0, i)),
            pl.BlockSpec((_BLK, H), lambda i: (i, 0)),
            pl.BlockSpec((1, H), lambda i: (0, 0)),
            pl.BlockSpec((1, H), lambda i: (0, 0)),
        ],
        out_specs=pl.BlockSpec((_BLK, H), lambda i: (i, 0)),
        out_shape=jax.ShapeDtypeStruct((NPAD, H), jnp.float32),
    )(agg, den, x, g, b)


def _tc3_body(x_ref, wq_ref, bq_ref, swr_ref, mask_ref, vec_ref, w_ref):
    x = x_ref[...]
    proj = jnp.tanh(jnp.dot(x, wq_ref[...],
                            preferred_element_type=jnp.float32) + bq_ref[...])
    sc = lax.dot_general(proj, swr_ref[...], (((1,), (1,)), ((), ())),
                         preferred_element_type=jnp.float32)[:, 0]
    sc = jnp.where(mask_ref[...] > 0.0, sc, -jnp.inf)
    m = jnp.max(sc)
    e = jnp.exp(sc - m)
    wgt = e / jnp.sum(e)
    wgt = jnp.where(jnp.isnan(wgt), 0.0, wgt)
    w_ref[...] = wgt
    gv = lax.dot_general(wgt, x, (((0,), (0,)), ((), ())),
                         preferred_element_type=jnp.float32)
    vec_ref[...] = gv[None, :]


def _tc3(x, Wq, bq, swr, mask):
    return pl.pallas_call(
        _tc3_body,
        out_shape=[
            jax.ShapeDtypeStruct((1, H), jnp.float32),
            jax.ShapeDtypeStruct((NPAD,), jnp.float32),
        ],
    )(x, Wq, bq, swr, mask)


def _gat_layer(x, src, dst, srcq, dstq, Wp, Wepair, g, b):
    h, s2 = _tc1(x, Wp, Wepair)
    scores, maxes = _k1(s2[:, 0], s2[:, 1], src, dst)
    agg, den = _k2(scores, maxes, srcq, dstq, h)
    return _tc2(agg, den, x, g, b)


def kernel(node_ids, edge_index, emb, W1p, W1e, g1, b1, W2p, W2e, g2, b2,
           Wq, bq, sw):
    f32 = jnp.float32
    ids = jnp.concatenate([node_ids.astype(jnp.int32),
                           jnp.zeros((NPAD - N,), jnp.int32)])
    idsq = ids.reshape(NW, 5, 64)
    src = edge_index[0].astype(jnp.int32)
    dst = edge_index[1].astype(jnp.int32)
    srcq = src.reshape(NW, NB, CPB, CH)
    dstq = dst.reshape(NW, NB, CPB, CH)
    mask = (ids != 0).astype(f32)

    x = _k0(emb, idsq)
    We1 = W1e[:, 0].reshape(2, H)
    We2 = W2e[:, 0].reshape(2, H)
    x = _gat_layer(x, src, dst, srcq, dstq, W1p, We1,
                   g1.reshape(1, H), b1.reshape(1, H))
    x = _gat_layer(x, src, dst, srcq, dstq, W2p, We2,
                   g2.reshape(1, H), b2.reshape(1, H))

    vec, wts = _tc3(x, Wq, bq.reshape(1, H), sw[:, 0].reshape(1, H), mask)
    return (vec, wts[:N], jnp.ones((1,), f32))
